# baseline (speedup 1.0000x reference)
import jax
import jax.numpy as jnp
from jax.experimental import pallas as pl
from jax.experimental.pallas import tpu as pltpu

_N_CORES = 2
_INNER = 5


def _copy_block(x_ref, o_ref):
    o_ref[...] = x_ref[...]


def kernel(x, edge_index):
    del edge_index
    n, d = x.shape
    block_rows = n // (_N_CORES * _INNER)
    return pl.pallas_call(
        _copy_block,
        grid=(_N_CORES, _INNER),
        in_specs=[pl.BlockSpec((block_rows, d), lambda i, j: (i * _INNER + j, 0))],
        out_specs=pl.BlockSpec((block_rows, d), lambda i, j: (i * _INNER + j, 0)),
        out_shape=jax.ShapeDtypeStruct((n, d), x.dtype),
        compiler_params=pltpu.CompilerParams(
            dimension_semantics=("parallel", "arbitrary"),
        ),
    )(x)


# speedup vs baseline: 1.9752x; 1.9752x over previous
import jax
import jax.numpy as jnp
from jax.experimental import pallas as pl
from jax.experimental.pallas import tpu as pltpu

_N_CHUNKS = 10


def _copy_overlap(x_hbm, o_hbm, buf, sem_in, sem_out):
    n = x_hbm.shape[0]
    c = n // _N_CHUNKS
    for i in range(_N_CHUNKS):
        pltpu.make_async_copy(
            x_hbm.at[pl.ds(i * c, c), :], buf.at[pl.ds(i * c, c), :], sem_in.at[i]
        ).start()
    for i in range(_N_CHUNKS):
        pltpu.make_async_copy(
            x_hbm.at[pl.ds(i * c, c), :], buf.at[pl.ds(i * c, c), :], sem_in.at[i]
        ).wait()
        pltpu.make_async_copy(
            buf.at[pl.ds(i * c, c), :], o_hbm.at[pl.ds(i * c, c), :], sem_out.at[i]
        ).start()
    for i in range(_N_CHUNKS):
        pltpu.make_async_copy(
            buf.at[pl.ds(i * c, c), :], o_hbm.at[pl.ds(i * c, c), :], sem_out.at[i]
        ).wait()


def kernel(x, edge_index):
    del edge_index
    n, d = x.shape
    return pl.pallas_call(
        _copy_overlap,
        in_specs=[pl.BlockSpec(memory_space=pl.ANY)],
        out_specs=pl.BlockSpec(memory_space=pl.ANY),
        out_shape=jax.ShapeDtypeStruct((n, d), x.dtype),
        scratch_shapes=[
            pltpu.VMEM((n, d), x.dtype),
            pltpu.SemaphoreType.DMA((_N_CHUNKS,)),
            pltpu.SemaphoreType.DMA((_N_CHUNKS,)),
        ],
    )(x)
